# baseline (speedup 1.0000x reference)
import functools

import jax
import jax.numpy as jnp
from jax.experimental import pallas as pl
from jax.experimental.pallas import tpu as pltpu

_EPS = 1e-5


def _stage_taps(x4, x9_ref, h, w):
    s, _, _, c = x4.shape
    xp = jnp.pad(x4, ((0, 0), (1, 1), (1, 1), (0, 0)))
    k = 0
    for dy in range(3):
        for dx in range(3):
            x9_ref[:, k * c:(k + 1) * c] = (
                xp[:, dy:dy + h, dx:dx + w, :].reshape(s * h * w, c))
            k += 1


def _acc_stats(y, ps, pq):
    m, c = y.shape
    y8 = y.reshape(m // 8, 8, c)
    return ps + jnp.sum(y8, axis=0), pq + jnp.sum(y8 * y8, axis=0)


def _affine(stat_row, g, beta, m_total):
    s = jnp.sum(stat_row[0], axis=0, keepdims=True)
    q = jnp.sum(stat_row[1], axis=0, keepdims=True)
    mean = s / m_total
    var = q / m_total - mean * mean
    sc = g * jax.lax.rsqrt(var + _EPS)
    sh = beta - mean * sc
    return sc, sh


def _fused_kernel(x_hbm, w_ref, bn_ref, down_hbm, pool_hbm,
                  ybuf, xbuf, obuf, pbuf, x9_ref, rp_ref, stat, aff,
                  insem, odsem, opsem,
                  *, h, w, blk, sub, nb, n, cin, c1, c2):
    hw = h * w
    p = pl.program_id(0)
    i = pl.program_id(1)
    m_total = float(n * hw)

    @pl.when((p == 0) & (i == 0))
    def _():
        stat[...] = jnp.zeros(stat.shape, stat.dtype)
        pltpu.make_async_copy(
            x_hbm.at[pl.ds(0, blk)], xbuf.at[0], insem.at[0]).start()

    @pl.when(p == 0)
    def _():
        @pl.when(i + 1 < nb)
        def _():
            s2 = (i + 1) % 2
            pltpu.make_async_copy(
                x_hbm.at[pl.ds((i + 1) * blk, blk)], xbuf.at[s2],
                insem.at[s2]).start()
        s1 = i % 2
        pltpu.make_async_copy(
            x_hbm.at[pl.ds(i * blk, blk)], xbuf.at[s1], insem.at[s1]).wait()
        ps = jnp.zeros((8, c1), jnp.float32)
        pq = jnp.zeros((8, c1), jnp.float32)
        for j in range(0, blk, sub):
            xs = xbuf[s1, j:j + sub]
            xt = jnp.swapaxes(xs, 1, 2).astype(jnp.bfloat16)
            _stage_taps(xt.reshape(sub, h, w, cin), x9_ref, h, w)
            y = jnp.dot(x9_ref[:, :9 * cin], w_ref[:9 * cin],
                        preferred_element_type=jnp.float32)
            ps, pq = _acc_stats(y, ps, pq)
            ybuf[pl.ds(i * blk + j, sub)] = (
                y.reshape(sub, hw, c1).astype(jnp.bfloat16))
        stat[0, 0] += ps
        stat[0, 1] += pq

    @pl.when((p == 1) & (i == 0))
    def _():
        sc1, sh1 = _affine(stat[0], bn_ref[0:1], bn_ref[1:2], m_total)
        aff[0:1] = sc1
        aff[1:2] = sh1

    @pl.when(p == 1)
    def _():
        ps = jnp.zeros((8, c2), jnp.float32)
        pq = jnp.zeros((8, c2), jnp.float32)
        for j in range(0, blk, sub):
            v = ybuf[pl.ds(i * blk + j, sub)].astype(jnp.float32)
            hr = jnp.maximum(v * aff[0:1] + aff[1:2], 0.0).astype(jnp.bfloat16)
            _stage_taps(hr.reshape(sub, h, w, c1), x9_ref, h, w)
            y = jnp.dot(x9_ref[...], w_ref[9 * cin:],
                        preferred_element_type=jnp.float32)
            ps, pq = _acc_stats(y, ps, pq)
            ybuf[pl.ds(i * blk + j, sub)] = (
                y.reshape(sub, hw, c2).astype(jnp.bfloat16))
        stat[1, 0] += ps
        stat[1, 1] += pq

    @pl.when((p == 2) & (i == 0))
    def _():
        sc2, sh2 = _affine(stat[1], bn_ref[2:3], bn_ref[3:4], m_total)
        aff[2:3] = sc2
        aff[3:4] = sh2

    @pl.when(p == 2)
    def _():
        slot = i % 2

        @pl.when(i >= 2)
        def _():
            pltpu.make_async_copy(
                obuf.at[slot], down_hbm.at[pl.ds((i - 2) * blk, blk)],
                odsem.at[slot]).wait()
            pltpu.make_async_copy(
                pbuf.at[slot], pool_hbm.at[pl.ds((i - 2) * blk, blk)],
                opsem.at[slot]).wait()

        v = ybuf[pl.ds(i * blk, blk)].astype(jnp.float32)
        d = jnp.maximum(v * aff[2:3] + aff[3:4], 0.0)
        obuf[slot] = jnp.swapaxes(d, 1, 2)
        half = d.reshape(blk * (h // 2), 2 * w, c2)
        rp = jnp.maximum(half[:, :w, :], half[:, w:, :])
        rp_ref[...] = rp.reshape(blk * (hw // 2), c2)
        npool = blk * (hw // 4)
        pr = jnp.maximum(rp_ref[pl.ds(0, npool, 2), :],
                         rp_ref[pl.ds(1, npool, 2), :])
        pbuf[slot] = jnp.swapaxes(pr.reshape(blk, hw // 4, c2), 1, 2)

        pltpu.make_async_copy(
            obuf.at[slot], down_hbm.at[pl.ds(i * blk, blk)],
            odsem.at[slot]).start()
        pltpu.make_async_copy(
            pbuf.at[slot], pool_hbm.at[pl.ds(i * blk, blk)],
            opsem.at[slot]).start()

        @pl.when(i == nb - 1)
        def _():
            for step in range(max(0, nb - 2), nb):
                s = step % 2
                pltpu.make_async_copy(
                    obuf.at[s], down_hbm.at[pl.ds(0, blk)], odsem.at[s]).wait()
                pltpu.make_async_copy(
                    pbuf.at[s], pool_hbm.at[pl.ds(0, blk)], opsem.at[s]).wait()


def kernel(x, w1, b1, g1, beta1, w2, b2, g2, beta2):
    n, cin, h, w = x.shape
    c1 = w1.shape[-1]
    c2 = w2.shape[-1]
    hw = h * w
    x3 = x.reshape(n, cin, hw)
    w1f = w1.reshape(9 * cin, c1).astype(jnp.bfloat16)
    w2f = w2.reshape(9 * c1, c2).astype(jnp.bfloat16)
    bn = jnp.concatenate([g1, beta1, g2, beta2], axis=0)
    wcat = jnp.concatenate([w1f, w2f], axis=0)

    blk = min(4, n)
    sub = min(2, blk)
    nb = n // blk

    down, pool = pl.pallas_call(
        functools.partial(_fused_kernel, h=h, w=w, blk=blk, sub=sub, nb=nb,
                          n=n, cin=cin, c1=c1, c2=c2),
        grid=(3, nb),
        in_specs=[
            pl.BlockSpec(memory_space=pltpu.MemorySpace.HBM),
            pl.BlockSpec((9 * cin + 9 * c1, c2), lambda p, i: (0, 0)),
            pl.BlockSpec((4, c1), lambda p, i: (0, 0)),
        ],
        out_specs=(
            pl.BlockSpec(memory_space=pltpu.MemorySpace.HBM),
            pl.BlockSpec(memory_space=pltpu.MemorySpace.HBM),
        ),
        out_shape=(
            jax.ShapeDtypeStruct((n, c2, hw), jnp.float32),
            jax.ShapeDtypeStruct((n, c2, hw // 4), jnp.float32),
        ),
        scratch_shapes=[
            pltpu.VMEM((n, hw, c1), jnp.bfloat16),
            pltpu.VMEM((2, blk, cin, hw), jnp.float32),
            pltpu.VMEM((2, blk, c2, hw), jnp.float32),
            pltpu.VMEM((2, blk, c2, hw // 4), jnp.float32),
            pltpu.VMEM((sub * hw, 9 * c1), jnp.bfloat16),
            pltpu.VMEM((blk * hw // 2, c2), jnp.float32),
            pltpu.VMEM((2, 2, 8, c2), jnp.float32),
            pltpu.VMEM((4, c2), jnp.float32),
            pltpu.SemaphoreType.DMA((2,)),
            pltpu.SemaphoreType.DMA((2,)),
            pltpu.SemaphoreType.DMA((2,)),
        ],
        compiler_params=pltpu.CompilerParams(
            dimension_semantics=("arbitrary", "arbitrary"),
            vmem_limit_bytes=60000 * 1024),
    )(x3, wcat, bn)
    return (down.reshape(n, c2, h, w), pool.reshape(n, c2, h // 2, w // 2))


# speedup vs baseline: 1.0449x; 1.0449x over previous
import functools

import jax
import jax.numpy as jnp
from jax.experimental import pallas as pl
from jax.experimental.pallas import tpu as pltpu

_EPS = 1e-5


def _stage_taps(x4, x9_ref, h, w):
    s, _, _, c = x4.shape
    xp = jnp.pad(x4, ((0, 0), (1, 1), (1, 1), (0, 0)))
    k = 0
    for dy in range(3):
        for dx in range(3):
            x9_ref[:, k * c:(k + 1) * c] = (
                xp[:, dy:dy + h, dx:dx + w, :].reshape(s * h * w, c))
            k += 1


def _acc_stats(y, ps, pq):
    m, c = y.shape
    y8 = y.reshape(m // 8, 8, c)
    return ps + jnp.sum(y8, axis=0), pq + jnp.sum(y8 * y8, axis=0)


def _affine(stat_row, g, beta, m_total):
    s = jnp.sum(stat_row[0], axis=0, keepdims=True)
    q = jnp.sum(stat_row[1], axis=0, keepdims=True)
    mean = s / m_total
    var = q / m_total - mean * mean
    sc = g * jax.lax.rsqrt(var + _EPS)
    sh = beta - mean * sc
    return sc, sh


def _fused_kernel(x_hbm, w_ref, bn_ref, down_hbm, pool_hbm,
                  ybuf, xbuf, obuf, pbuf, x9_ref, rp_ref, stat, aff,
                  insem, odsem, opsem,
                  *, h, w, blk, sub, nb, n, cin, c1, c2):
    hw = h * w
    p = pl.program_id(0)
    i = pl.program_id(1)
    m_total = float(n * hw)

    @pl.when((p == 0) & (i == 0))
    def _():
        stat[...] = jnp.zeros(stat.shape, stat.dtype)
        pltpu.make_async_copy(
            x_hbm.at[pl.ds(0, blk)], xbuf.at[0], insem.at[0]).start()

    @pl.when(p == 0)
    def _():
        @pl.when(i + 1 < nb)
        def _():
            s2 = (i + 1) % 2
            pltpu.make_async_copy(
                x_hbm.at[pl.ds((i + 1) * blk, blk)], xbuf.at[s2],
                insem.at[s2]).start()
        s1 = i % 2
        pltpu.make_async_copy(
            x_hbm.at[pl.ds(i * blk, blk)], xbuf.at[s1], insem.at[s1]).wait()
        ps = jnp.zeros((8, c1), jnp.float32)
        pq = jnp.zeros((8, c1), jnp.float32)
        for j in range(0, blk, sub):
            xs = xbuf[s1, j:j + sub]
            xt = jnp.swapaxes(xs, 1, 2).astype(jnp.bfloat16)
            _stage_taps(xt.reshape(sub, h, w, cin), x9_ref, h, w)
            y = jnp.dot(x9_ref[:, :9 * cin], w_ref[:9 * cin],
                        preferred_element_type=jnp.float32)
            ps, pq = _acc_stats(y, ps, pq)
            ybuf[pl.ds(i * blk + j, sub)] = (
                y.reshape(sub, hw, c1).astype(jnp.bfloat16))
        stat[0, 0] += ps
        stat[0, 1] += pq

    @pl.when((p == 1) & (i == 0))
    def _():
        sc1, sh1 = _affine(stat[0], bn_ref[0:1], bn_ref[1:2], m_total)
        aff[0:1] = sc1
        aff[1:2] = sh1

    @pl.when(p == 1)
    def _():
        ps = jnp.zeros((8, c2), jnp.float32)
        pq = jnp.zeros((8, c2), jnp.float32)
        for j in range(0, blk, sub):
            v = ybuf[pl.ds(i * blk + j, sub)].astype(jnp.float32)
            hr = jnp.maximum(v * aff[0:1] + aff[1:2], 0.0).astype(jnp.bfloat16)
            _stage_taps(hr.reshape(sub, h, w, c1), x9_ref, h, w)
            y = jnp.dot(x9_ref[...], w_ref[9 * cin:],
                        preferred_element_type=jnp.float32)
            ps, pq = _acc_stats(y, ps, pq)
            ybuf[pl.ds(i * blk + j, sub)] = (
                y.reshape(sub, hw, c2).astype(jnp.bfloat16))
        stat[1, 0] += ps
        stat[1, 1] += pq

    @pl.when((p == 2) & (i == 0))
    def _():
        sc2, sh2 = _affine(stat[1], bn_ref[2:3], bn_ref[3:4], m_total)
        aff[2:3] = sc2
        aff[3:4] = sh2

    @pl.when(p == 2)
    def _():
        slot = i % 2

        @pl.when(i >= 2)
        def _():
            pltpu.make_async_copy(
                obuf.at[slot], down_hbm.at[pl.ds((i - 2) * blk, blk)],
                odsem.at[slot]).wait()
            pltpu.make_async_copy(
                pbuf.at[slot], pool_hbm.at[pl.ds((i - 2) * blk, blk)],
                opsem.at[slot]).wait()

        v = ybuf[pl.ds(i * blk, blk)].astype(jnp.float32)
        d = jnp.maximum(v * aff[2:3] + aff[3:4], 0.0)
        obuf[slot] = jnp.swapaxes(d, 1, 2)
        bh = blk // 2
        for t in range(2):
            dt = d[t * bh:(t + 1) * bh]
            half = dt.reshape(bh * (h // 2), 2 * w, c2)
            rp = jnp.maximum(half[:, :w, :], half[:, w:, :])
            rp_ref[...] = rp.reshape(bh * (hw // 2), c2)
            npool = bh * (hw // 4)
            pr = jnp.maximum(rp_ref[pl.ds(0, npool, 2), :],
                             rp_ref[pl.ds(1, npool, 2), :])
            pbuf[slot, t * bh:(t + 1) * bh] = (
                jnp.swapaxes(pr.reshape(bh, hw // 4, c2), 1, 2))

        pltpu.make_async_copy(
            obuf.at[slot], down_hbm.at[pl.ds(i * blk, blk)],
            odsem.at[slot]).start()
        pltpu.make_async_copy(
            pbuf.at[slot], pool_hbm.at[pl.ds(i * blk, blk)],
            opsem.at[slot]).start()

        @pl.when(i == nb - 1)
        def _():
            for step in range(max(0, nb - 2), nb):
                s = step % 2
                pltpu.make_async_copy(
                    obuf.at[s], down_hbm.at[pl.ds(0, blk)], odsem.at[s]).wait()
                pltpu.make_async_copy(
                    pbuf.at[s], pool_hbm.at[pl.ds(0, blk)], opsem.at[s]).wait()


def kernel(x, w1, b1, g1, beta1, w2, b2, g2, beta2):
    n, cin, h, w = x.shape
    c1 = w1.shape[-1]
    c2 = w2.shape[-1]
    hw = h * w
    x3 = x.reshape(n, cin, hw)
    w1f = w1.reshape(9 * cin, c1).astype(jnp.bfloat16)
    w2f = w2.reshape(9 * c1, c2).astype(jnp.bfloat16)
    bn = jnp.concatenate([g1, beta1, g2, beta2], axis=0)
    wcat = jnp.concatenate([w1f, w2f], axis=0)

    blk = min(8, n)
    sub = 1
    nb = n // blk

    down, pool = pl.pallas_call(
        functools.partial(_fused_kernel, h=h, w=w, blk=blk, sub=sub, nb=nb,
                          n=n, cin=cin, c1=c1, c2=c2),
        grid=(3, nb),
        in_specs=[
            pl.BlockSpec(memory_space=pltpu.MemorySpace.HBM),
            pl.BlockSpec((9 * cin + 9 * c1, c2), lambda p, i: (0, 0)),
            pl.BlockSpec((4, c1), lambda p, i: (0, 0)),
        ],
        out_specs=(
            pl.BlockSpec(memory_space=pltpu.MemorySpace.HBM),
            pl.BlockSpec(memory_space=pltpu.MemorySpace.HBM),
        ),
        out_shape=(
            jax.ShapeDtypeStruct((n, c2, hw), jnp.float32),
            jax.ShapeDtypeStruct((n, c2, hw // 4), jnp.float32),
        ),
        scratch_shapes=[
            pltpu.VMEM((n, hw, c1), jnp.bfloat16),
            pltpu.VMEM((2, blk, cin, hw), jnp.float32),
            pltpu.VMEM((2, blk, c2, hw), jnp.float32),
            pltpu.VMEM((2, blk, c2, hw // 4), jnp.float32),
            pltpu.VMEM((sub * hw, 9 * c1), jnp.bfloat16),
            pltpu.VMEM((blk * hw // 4, c2), jnp.float32),
            pltpu.VMEM((2, 2, 8, c2), jnp.float32),
            pltpu.VMEM((4, c2), jnp.float32),
            pltpu.SemaphoreType.DMA((2,)),
            pltpu.SemaphoreType.DMA((2,)),
            pltpu.SemaphoreType.DMA((2,)),
        ],
        compiler_params=pltpu.CompilerParams(
            dimension_semantics=("arbitrary", "arbitrary"),
            vmem_limit_bytes=60000 * 1024),
    )(x3, wcat, bn)
    return (down.reshape(n, c2, h, w), pool.reshape(n, c2, h // 2, w // 2))
